# baseline (speedup 1.0000x reference)
import functools

import jax
import jax.numpy as jnp
from jax import lax
from jax.experimental import pallas as pl
from jax.experimental.pallas import tpu as pltpu
from jax.experimental.pallas import tpu_sc as plsc

N_USERS = 25000
N_ITEMS = 25000
EMB = 128
E = 500000
N = (N_USERS + 1) + (N_ITEMS + 1)

NPAD = 50176
BLK = 512
GRID = NPAD // BLK

NCORES = 2
NTILES = 16
RPT = NPAD // NTILES
OROWS = RPT // 4
BW = 128
NB = 245
G = 7
NG = NB // G
GBW = G * BW
EPT = NB * BW
EP = NTILES * EPT
HW = 8
NDUMMY = NPAD - N

_MESH = plsc.VectorSubcoreMesh(
    core_axis_name="c", subcore_axis_name="s",
    num_cores=NCORES, num_subcores=NTILES)
_SC_PARAMS = pltpu.CompilerParams(use_tc_tiling_on_sc=False)


@functools.partial(
    pl.kernel,
    out_type=jax.ShapeDtypeStruct((2, 6, NPAD, HW), jnp.float32),
    mesh=_MESH,
    scratch_types=[
        pltpu.VMEM((EPT,), jnp.int32),
        pltpu.VMEM((GBW, HW), jnp.float32),
        pltpu.VMEM((OROWS, HW), jnp.float32),
        pltpu.VMEM((OROWS, HW), jnp.float32),
        pltpu.VMEM_SHARED((NPAD, HW), jnp.float32),
    ],
    compiler_params=_SC_PARAMS,
)
def _hist_sc(edges_hbm, ones_hbm, zeros_hbm, degs_hbm, ids, ones, zbuf, obuf, acc):
    c = lax.axis_index("c")
    s = lax.axis_index("s")
    row0 = s * RPT

    pltpu.sync_copy(ones_hbm, ones)
    pltpu.sync_copy(zeros_hbm, zbuf)

    for b in range(6):
        pltpu.sync_copy(edges_hbm.at[b, c, s], ids)
        for z in range(RPT // OROWS):
            pltpu.sync_copy(zbuf, acc.at[pl.ds(row0 + z * OROWS, OROWS)])
        plsc.subcore_barrier()

        def body(j, _):
            pltpu.sync_copy(ones, acc.at[ids.at[pl.ds(j * GBW, GBW)]], add=True)
            return 0

        lax.fori_loop(0, NG, body, 0)
        plsc.subcore_barrier()
        for z in range(RPT // OROWS):
            pltpu.sync_copy(acc.at[pl.ds(row0 + z * OROWS, OROWS)], obuf)
            pltpu.sync_copy(obuf, degs_hbm.at[c, b, pl.ds(row0 + z * OROWS, OROWS)])
        plsc.subcore_barrier()


def _make_conv(nbeh, nchunk):
    cw = EMB // nchunk

    @functools.partial(
        pl.kernel,
        out_type=jax.ShapeDtypeStruct((nbeh, nchunk, NPAD, cw), jnp.float32),
        mesh=_MESH,
        scratch_types=[
            pltpu.VMEM((EPT,), jnp.int32),
            pltpu.VMEM((EPT,), jnp.int32),
            pltpu.VMEM((GBW, cw), jnp.float32),
            pltpu.VMEM((GBW, cw), jnp.float32),
            pltpu.VMEM((OROWS, cw), jnp.float32),
            pltpu.VMEM((OROWS, cw), jnp.float32),
            pltpu.VMEM_SHARED((NPAD, cw), jnp.float32),
            pltpu.SemaphoreType.DMA,
            pltpu.SemaphoreType.DMA,
        ],
        compiler_params=_SC_PARAMS,
    )
    def conv(srcn_hbm, dst_hbm, xsflat_hbm, zeros_hbm, y_hbm,
             gidx, didx, rows0, rows1, zbuf, obuf, acc, sem0, sem1):
        c = lax.axis_index("c")
        s = lax.axis_index("s")
        row0 = s * RPT

        pltpu.sync_copy(zeros_hbm, zbuf)

        for b in range(nbeh):
            pltpu.sync_copy(dst_hbm.at[b, s], didx)
            for p in range(nchunk // NCORES):
                cc = c * (nchunk // NCORES) + p
                pltpu.sync_copy(srcn_hbm.at[b, cc, s], gidx)
                for z in range(RPT // OROWS):
                    pltpu.sync_copy(zbuf, acc.at[pl.ds(row0 + z * OROWS, OROWS)])
                plsc.subcore_barrier()

                def gat(j):
                    return xsflat_hbm.at[gidx.at[pl.ds(j * GBW, GBW)]]

                def sca(buf, j):
                    pltpu.sync_copy(buf, acc.at[didx.at[pl.ds(j * GBW, GBW)]],
                                    add=True)

                pltpu.async_copy(gat(0), rows0, sem0)
                pltpu.async_copy(gat(1), rows1, sem1)

                def body(i, _):
                    j0 = 2 * i
                    pltpu.make_async_copy(gat(j0), rows0, sem0).wait()
                    sca(rows0, j0)

                    @pl.when(j0 + 2 < NG)
                    def _():
                        pltpu.async_copy(gat(j0 + 2), rows0, sem0)

                    pltpu.make_async_copy(gat(j0 + 1), rows1, sem1).wait()
                    sca(rows1, j0 + 1)

                    @pl.when(j0 + 3 < NG)
                    def _():
                        pltpu.async_copy(gat(j0 + 3), rows1, sem1)

                    return 0

                lax.fori_loop(0, NG // 2, body, 0)
                pltpu.make_async_copy(gat(NG - 1), rows0, sem0).wait()
                sca(rows0, NG - 1)

                plsc.subcore_barrier()
                for z in range(RPT // OROWS):
                    pltpu.sync_copy(acc.at[pl.ds(row0 + z * OROWS, OROWS)], obuf)
                    pltpu.sync_copy(obuf, y_hbm.at[b, cc, pl.ds(row0 + z * OROWS, OROWS)])
                plsc.subcore_barrier()

    return conv


_conv_a = _make_conv(1, 16)
_conv_b = _make_conv(3, 16)
_conv_c = _make_conv(2, 16)


def _pre_body(x_ref, deg_ref, o_ref):
    a = lax.rsqrt(jnp.maximum(deg_ref[:, 0:1], 1.0))
    o_ref[...] = x_ref[...] * a


_pre_tc = pl.pallas_call(
    _pre_body,
    grid=(GRID,),
    in_specs=[
        pl.BlockSpec((BLK, EMB), lambda i: (i, 0)),
        pl.BlockSpec((BLK, HW), lambda i: (i, 0)),
    ],
    out_specs=pl.BlockSpec((BLK, EMB), lambda i: (i, 0)),
    out_shape=jax.ShapeDtypeStruct((NPAD, EMB), jnp.float32),
)


def _post_body(y_ref, x_ref, deg_ref, o_ref):
    b = lax.rsqrt(jnp.maximum(deg_ref[:, 0:1], 1.0))
    t = y_ref[...] * b
    n = jnp.sqrt(jnp.sum(t * t, axis=1, keepdims=True))
    o_ref[...] = x_ref[...] + t / jnp.maximum(n, 1e-12)


_post_tc = pl.pallas_call(
    _post_body,
    grid=(GRID,),
    in_specs=[
        pl.BlockSpec((BLK, EMB), lambda i: (i, 0)),
        pl.BlockSpec((BLK, EMB), lambda i: (i, 0)),
        pl.BlockSpec((BLK, HW), lambda i: (i, 0)),
    ],
    out_specs=pl.BlockSpec((BLK, EMB), lambda i: (i, 0)),
    out_shape=jax.ShapeDtypeStruct((NPAD, EMB), jnp.float32),
)


def _fuse_body(w_ref, e0, e1, e2, e3, e4, e5, proj_ref, o_ref):
    acc = w_ref[0] * e0[...]
    for i, e in enumerate((e1, e2, e3, e4, e5)):
        acc = acc + w_ref[i + 1] * e[...]
    o_ref[...] = jnp.dot(acc, proj_ref[...],
                         preferred_element_type=jnp.float32)


_fuse_tc = pl.pallas_call(
    _fuse_body,
    grid=(GRID,),
    in_specs=[pl.BlockSpec(memory_space=pltpu.SMEM)]
    + [pl.BlockSpec((BLK, EMB), lambda i: (i, 0)) for _ in range(6)]
    + [pl.BlockSpec((EMB, EMB), lambda i: (0, 0))],
    out_specs=pl.BlockSpec((BLK, EMB), lambda i: (i, 0)),
    out_shape=jax.ShapeDtypeStruct((NPAD, EMB), jnp.float32),
)


def _conv_group(conv_fn, nchunk, xs_list, packs_sel, zeros):
    cw = EMB // nchunk
    nbeh = len(xs_list)
    srcn = jnp.stack([p[0] for p in packs_sel])
    dstn = jnp.stack([p[1] for p in packs_sel])
    xsflat = jnp.stack(xs_list).reshape(nbeh * NPAD * nchunk, cw)
    y = conv_fn(srcn, dstn, xsflat, zeros)
    outs = []
    for b in range(nbeh):
        outs.append(jnp.concatenate([y[b, i] for i in range(nchunk)], axis=1))
    return outs


def kernel(user_table, item_table, fusion_w, fusion_proj,
           edge_ubg, edge_view, edge_cart, edge_buy,
           edge_view_buy, edge_cart_buy):
    edges = [edge_ubg, edge_view, edge_cart, edge_buy,
             edge_view_buy, edge_cart_buy]
    edges = [e.astype(jnp.int32) for e in edges]

    x0 = jnp.concatenate([user_table, item_table], axis=0)
    x0 = jnp.pad(x0, ((0, NPAD - N), (0, 0)))

    pad_ids = (N + jnp.arange(EP - E, dtype=jnp.int32) % NDUMMY)[None, :]
    pad_ids = jnp.concatenate([pad_ids, pad_ids], axis=0)
    arange8 = jnp.arange(8, dtype=jnp.int32)
    arange16 = jnp.arange(16, dtype=jnp.int32)

    def make_pack(e, beh_in_launch, nchunk):
        ar = arange8 if nchunk == 8 else arange16
        ep = jnp.concatenate([e, pad_ids], axis=1)
        base = (beh_in_launch * NPAD + ep[0]) * nchunk
        srcn = base[None, :] + ar[:, None]
        return (srcn.reshape(nchunk, NTILES, EPT),
                ep[1].reshape(NTILES, EPT),
                ep.reshape(2, NTILES, EPT))

    packs = [make_pack(edges[0], 0, 16),
             make_pack(edges[1], 0, 16),
             make_pack(edges[2], 1, 16),
             make_pack(edges[3], 2, 16),
             make_pack(edges[4], 0, 16),
             make_pack(edges[5], 1, 16)]
    edges_all = jnp.stack([p[2] for p in packs])

    ones8 = jnp.ones((GBW, HW), jnp.float32)
    zeros8 = jnp.zeros((OROWS, HW), jnp.float32)
    degs = _hist_sc(edges_all, ones8, zeros8)

    xs0 = _pre_tc(x0, degs[0, 0])
    (y_ubg,) = _conv_group(_conv_a, 16, [xs0], packs[0:1], zeros8)
    emb_ubg = _post_tc(y_ubg, x0, degs[1, 0])

    xs_b = [_pre_tc(emb_ubg, degs[0, i]) for i in (1, 2, 3)]
    y_view, y_cart, y_buy = _conv_group(_conv_b, 16, xs_b, packs[1:4], zeros8)
    emb_view = _post_tc(y_view, emb_ubg, degs[1, 1])
    emb_cart = _post_tc(y_cart, emb_ubg, degs[1, 2])
    emb_buy = _post_tc(y_buy, emb_ubg, degs[1, 3])

    xs_c = [_pre_tc(emb_view, degs[0, 4]), _pre_tc(emb_cart, degs[0, 5])]
    y_vb, y_cb = _conv_group(_conv_c, 16, xs_c, packs[4:6], zeros8)
    emb_vb = _post_tc(y_vb, emb_view, degs[1, 4])
    emb_cb = _post_tc(y_cb, emb_cart, degs[1, 5])

    w = jax.nn.softmax(fusion_w)
    fused = _fuse_tc(w, emb_ubg, emb_view, emb_cart, emb_buy,
                     emb_vb, emb_cb, fusion_proj)
    return fused[:N]


# speedup vs baseline: 1.2845x; 1.2845x over previous
import functools

import jax
import jax.numpy as jnp
from jax import lax
from jax.experimental import pallas as pl
from jax.experimental.pallas import tpu as pltpu
from jax.experimental.pallas import tpu_sc as plsc

N_USERS = 25000
N_ITEMS = 25000
EMB = 128
E = 500000
N = (N_USERS + 1) + (N_ITEMS + 1)

NPAD = 50176
BLK = 512
GRID = NPAD // BLK

NCORES = 2
NTILES = 16
RPT = NPAD // NTILES
OROWS = RPT // 4
BW = 128
NB = 245
EPT = NB * BW
EP = NTILES * EPT
NCHUNK = 8
CW = EMB // NCHUNK
HW = 8
NDUMMY = NPAD - N

_MESH = plsc.VectorSubcoreMesh(
    core_axis_name="c", subcore_axis_name="s",
    num_cores=NCORES, num_subcores=NTILES)
_SC_PARAMS = pltpu.CompilerParams(use_tc_tiling_on_sc=False)


@functools.partial(
    pl.kernel,
    out_type=jax.ShapeDtypeStruct((2, 6, NPAD, HW), jnp.float32),
    mesh=_MESH,
    scratch_types=[
        pltpu.VMEM((NB, BW), jnp.int32),
        pltpu.VMEM((BW, HW), jnp.float32),
        pltpu.VMEM((OROWS, HW), jnp.float32),
        pltpu.VMEM((OROWS, HW), jnp.float32),
        pltpu.VMEM_SHARED((NPAD, HW), jnp.float32),
    ],
    compiler_params=_SC_PARAMS,
)
def _hist_sc(edges_hbm, ones_hbm, zeros_hbm, degs_hbm, ids, ones, zbuf, obuf, acc):
    c = lax.axis_index("c")
    s = lax.axis_index("s")
    row0 = s * RPT

    pltpu.sync_copy(ones_hbm, ones)
    pltpu.sync_copy(zeros_hbm, zbuf)

    for b in range(6):
        pltpu.sync_copy(edges_hbm.at[b, c, s], ids)
        for z in range(RPT // OROWS):
            pltpu.sync_copy(zbuf, acc.at[pl.ds(row0 + z * OROWS, OROWS)])
        plsc.subcore_barrier()

        def body(j, _):
            pltpu.sync_copy(ones, acc.at[ids.at[j]], add=True)
            return 0

        lax.fori_loop(0, NB, body, 0)
        plsc.subcore_barrier()
        for z in range(RPT // OROWS):
            pltpu.sync_copy(acc.at[pl.ds(row0 + z * OROWS, OROWS)], obuf)
            pltpu.sync_copy(obuf, degs_hbm.at[c, b, pl.ds(row0 + z * OROWS, OROWS)])
        plsc.subcore_barrier()


@functools.partial(
    pl.kernel,
    out_type=jax.ShapeDtypeStruct((NCHUNK, NPAD, CW), jnp.bfloat16),
    mesh=_MESH,
    scratch_types=[
        pltpu.VMEM((NB, BW), jnp.int32),
        pltpu.VMEM((NB, BW), jnp.int32),
        pltpu.VMEM((BW, CW), jnp.bfloat16),
        pltpu.VMEM((BW, CW), jnp.bfloat16),
        pltpu.VMEM((OROWS, CW), jnp.bfloat16),
        pltpu.VMEM((OROWS, CW), jnp.bfloat16),
        pltpu.VMEM_SHARED((NPAD, CW), jnp.bfloat16),
        pltpu.SemaphoreType.DMA,
        pltpu.SemaphoreType.DMA,
    ],
    compiler_params=_SC_PARAMS,
)
def _conv_sc(srcn_hbm, dst_hbm, xsflat_hbm, y_hbm,
             gidx, didx, rows0, rows1, zbuf, obuf, acc, sem0, sem1):
    c = lax.axis_index("c")
    s = lax.axis_index("s")
    row0 = s * RPT

    zero216 = jnp.zeros((2, 16), jnp.bfloat16)

    def zrow(i, _):
        zbuf[pl.ds(i * 2, 2), :] = zero216
        return 0

    lax.fori_loop(0, OROWS // 2, zrow, 0)
    pltpu.sync_copy(dst_hbm.at[s], didx)

    for p in range(NCHUNK // NCORES):
        cc = c * (NCHUNK // NCORES) + p
        pltpu.sync_copy(srcn_hbm.at[cc, s], gidx)
        for z in range(RPT // OROWS):
            pltpu.sync_copy(zbuf, acc.at[pl.ds(row0 + z * OROWS, OROWS)])
        plsc.subcore_barrier()

        pltpu.async_copy(xsflat_hbm.at[gidx.at[0]], rows0, sem0)
        pltpu.async_copy(xsflat_hbm.at[gidx.at[1]], rows1, sem1)

        def body(i, _):
            j0 = 2 * i
            pltpu.make_async_copy(xsflat_hbm.at[gidx.at[j0]], rows0, sem0).wait()
            pltpu.sync_copy(rows0, acc.at[didx.at[j0]], add=True)

            @pl.when(j0 + 2 < NB)
            def _():
                pltpu.async_copy(xsflat_hbm.at[gidx.at[j0 + 2]], rows0, sem0)

            pltpu.make_async_copy(xsflat_hbm.at[gidx.at[j0 + 1]], rows1, sem1).wait()
            pltpu.sync_copy(rows1, acc.at[didx.at[j0 + 1]], add=True)

            @pl.when(j0 + 3 < NB)
            def _():
                pltpu.async_copy(xsflat_hbm.at[gidx.at[j0 + 3]], rows1, sem1)

            return 0

        lax.fori_loop(0, NB // 2, body, 0)
        pltpu.make_async_copy(xsflat_hbm.at[gidx.at[NB - 1]], rows0, sem0).wait()
        pltpu.sync_copy(rows0, acc.at[didx.at[NB - 1]], add=True)

        plsc.subcore_barrier()
        for z in range(RPT // OROWS):
            pltpu.sync_copy(acc.at[pl.ds(row0 + z * OROWS, OROWS)], obuf)
            pltpu.sync_copy(obuf, y_hbm.at[cc, pl.ds(row0 + z * OROWS, OROWS)])
        plsc.subcore_barrier()


def _pre_body(x_ref, deg_ref, o_ref):
    a = lax.rsqrt(jnp.maximum(deg_ref[:, 0:1], 1.0))
    o_ref[...] = (x_ref[...] * a).astype(jnp.bfloat16)


_pre_tc = pl.pallas_call(
    _pre_body,
    grid=(GRID,),
    in_specs=[
        pl.BlockSpec((BLK, EMB), lambda i: (i, 0)),
        pl.BlockSpec((BLK, HW), lambda i: (i, 0)),
    ],
    out_specs=pl.BlockSpec((BLK, EMB), lambda i: (i, 0)),
    out_shape=jax.ShapeDtypeStruct((NPAD, EMB), jnp.bfloat16),
)


def _post_body(y_ref, x_ref, deg_ref, o_ref):
    b = lax.rsqrt(jnp.maximum(deg_ref[:, 0:1], 1.0))
    t = y_ref[...].astype(jnp.float32) * b
    n = jnp.sqrt(jnp.sum(t * t, axis=1, keepdims=True))
    o_ref[...] = x_ref[...] + t / jnp.maximum(n, 1e-12)


_post_tc = pl.pallas_call(
    _post_body,
    grid=(GRID,),
    in_specs=[
        pl.BlockSpec((BLK, EMB), lambda i: (i, 0)),
        pl.BlockSpec((BLK, EMB), lambda i: (i, 0)),
        pl.BlockSpec((BLK, HW), lambda i: (i, 0)),
    ],
    out_specs=pl.BlockSpec((BLK, EMB), lambda i: (i, 0)),
    out_shape=jax.ShapeDtypeStruct((NPAD, EMB), jnp.float32),
)


def _fuse_body(w_ref, e0, e1, e2, e3, e4, e5, proj_ref, o_ref):
    acc = w_ref[0] * e0[...]
    for i, e in enumerate((e1, e2, e3, e4, e5)):
        acc = acc + w_ref[i + 1] * e[...]
    o_ref[...] = jnp.dot(acc, proj_ref[...],
                         preferred_element_type=jnp.float32)


_fuse_tc = pl.pallas_call(
    _fuse_body,
    grid=(GRID,),
    in_specs=[pl.BlockSpec(memory_space=pltpu.SMEM)]
    + [pl.BlockSpec((BLK, EMB), lambda i: (i, 0)) for _ in range(6)]
    + [pl.BlockSpec((EMB, EMB), lambda i: (0, 0))],
    out_specs=pl.BlockSpec((BLK, EMB), lambda i: (i, 0)),
    out_shape=jax.ShapeDtypeStruct((NPAD, EMB), jnp.float32),
)


def _conv_step(x, edge_pack, deg_out, deg_in):
    srcn, dstn = edge_pack[0], edge_pack[1]
    xs = _pre_tc(x, deg_out)
    xs_flat = xs.reshape(NPAD * NCHUNK, CW)
    y = _conv_sc(srcn, dstn, xs_flat)
    y = jnp.concatenate([y[i] for i in range(NCHUNK)], axis=1)
    return _post_tc(y, x, deg_in)


def kernel(user_table, item_table, fusion_w, fusion_proj,
           edge_ubg, edge_view, edge_cart, edge_buy,
           edge_view_buy, edge_cart_buy):
    edges = [edge_ubg, edge_view, edge_cart, edge_buy,
             edge_view_buy, edge_cart_buy]
    edges = [e.astype(jnp.int32) for e in edges]

    x0 = jnp.concatenate([user_table, item_table], axis=0)
    x0 = jnp.pad(x0, ((0, NPAD - N), (0, 0)))

    pad_ids = (N + jnp.arange(EP - E, dtype=jnp.int32) % NDUMMY)[None, :]
    pad_ids = jnp.concatenate([pad_ids, pad_ids], axis=0)
    packs = []
    for e in edges:
        ep = jnp.concatenate([e, pad_ids], axis=1)
        srcn = (ep[0] * NCHUNK)[None, :] + jnp.arange(NCHUNK, dtype=jnp.int32)[:, None]
        packs.append((srcn.reshape(NCHUNK, NTILES, NB, BW),
                      ep[1].reshape(NTILES, NB, BW),
                      ep.reshape(2, NTILES, NB, BW)))
    edges_all = jnp.stack([p[2] for p in packs])

    ones8 = jnp.ones((BW, HW), jnp.float32)
    zeros8 = jnp.zeros((OROWS, HW), jnp.float32)
    degs = _hist_sc(edges_all, ones8, zeros8)

    def _after(x, prev):
        return lax.optimization_barrier((x, prev))[0]

    emb_ubg = _conv_step(x0, packs[0], degs[0, 0], degs[1, 0])
    emb_view = _conv_step(emb_ubg, packs[1], degs[0, 1], degs[1, 1])
    emb_cart = _conv_step(_after(emb_ubg, emb_view), packs[2],
                          degs[0, 2], degs[1, 2])
    emb_buy = _conv_step(_after(emb_ubg, emb_cart), packs[3],
                         degs[0, 3], degs[1, 3])
    emb_vb = _conv_step(_after(emb_view, emb_buy), packs[4],
                        degs[0, 4], degs[1, 4])
    emb_cb = _conv_step(_after(emb_cart, emb_vb), packs[5],
                        degs[0, 5], degs[1, 5])

    w = jax.nn.softmax(fusion_w)
    fused = _fuse_tc(w, emb_ubg, emb_view, emb_cart, emb_buy,
                     emb_vb, emb_cb, fusion_proj)
    return fused[:N]


# speedup vs baseline: 1.7427x; 1.3567x over previous
import functools

import jax
import jax.numpy as jnp
from jax import lax
from jax.experimental import pallas as pl
from jax.experimental.pallas import tpu as pltpu
from jax.experimental.pallas import tpu_sc as plsc

N_USERS = 25000
N_ITEMS = 25000
EMB = 128
E = 500000
N = (N_USERS + 1) + (N_ITEMS + 1)

NPAD = 50176
BLK = 512
GRID = NPAD // BLK

NCORES = 2
NTILES = 16
RPT = NPAD // NTILES
OROWS = RPT // 4
BW = 128
NB = 245
G = 7
NG = NB // G
GBW = G * BW
EPT = NB * BW
EP = NTILES * EPT
NCHUNK = 8
CW = EMB // NCHUNK
HW = 8
NDUMMY = NPAD - N

_MESH = plsc.VectorSubcoreMesh(
    core_axis_name="c", subcore_axis_name="s",
    num_cores=NCORES, num_subcores=NTILES)
_SC_PARAMS = pltpu.CompilerParams(use_tc_tiling_on_sc=False)


@functools.partial(
    pl.kernel,
    out_type=jax.ShapeDtypeStruct((2, 6, NPAD, HW), jnp.float32),
    mesh=_MESH,
    scratch_types=[
        pltpu.VMEM((EPT,), jnp.int32),
        pltpu.VMEM((GBW, HW), jnp.float32),
        pltpu.VMEM((OROWS, HW), jnp.float32),
        pltpu.VMEM((OROWS, HW), jnp.float32),
        pltpu.VMEM_SHARED((NPAD, HW), jnp.float32),
    ],
    compiler_params=_SC_PARAMS,
)
def _hist_sc(edges_hbm, ones_hbm, zeros_hbm, degs_hbm, ids, ones, zbuf, obuf, acc):
    c = lax.axis_index("c")
    s = lax.axis_index("s")
    row0 = s * RPT

    pltpu.sync_copy(ones_hbm, ones)
    pltpu.sync_copy(zeros_hbm, zbuf)

    for b in range(6):
        pltpu.sync_copy(edges_hbm.at[b, c, s], ids)
        for z in range(RPT // OROWS):
            pltpu.sync_copy(zbuf, acc.at[pl.ds(row0 + z * OROWS, OROWS)])
        plsc.subcore_barrier()

        def body(j, _):
            pltpu.sync_copy(ones, acc.at[ids.at[pl.ds(j * GBW, GBW)]], add=True)
            return 0

        lax.fori_loop(0, NG, body, 0)
        plsc.subcore_barrier()
        for z in range(RPT // OROWS):
            pltpu.sync_copy(acc.at[pl.ds(row0 + z * OROWS, OROWS)], obuf)
            pltpu.sync_copy(obuf, degs_hbm.at[c, b, pl.ds(row0 + z * OROWS, OROWS)])
        plsc.subcore_barrier()


def _make_conv(nbeh):
    @functools.partial(
        pl.kernel,
        out_type=jax.ShapeDtypeStruct((nbeh, NCHUNK, NPAD, CW), jnp.bfloat16),
        mesh=_MESH,
        scratch_types=[
            pltpu.VMEM((EPT,), jnp.int32),
            pltpu.VMEM((EPT,), jnp.int32),
            pltpu.VMEM((GBW, CW), jnp.bfloat16),
            pltpu.VMEM((GBW, CW), jnp.bfloat16),
            pltpu.VMEM((OROWS, CW), jnp.bfloat16),
            pltpu.VMEM((OROWS, CW), jnp.bfloat16),
            pltpu.VMEM_SHARED((NPAD, CW), jnp.bfloat16),
            pltpu.SemaphoreType.DMA,
            pltpu.SemaphoreType.DMA,
        ],
        compiler_params=_SC_PARAMS,
    )
    def conv(srcn_hbm, dst_hbm, xsflat_hbm, y_hbm,
             gidx, didx, rows0, rows1, zbuf, obuf, acc, sem0, sem1):
        c = lax.axis_index("c")
        s = lax.axis_index("s")
        row0 = s * RPT

        zero216 = jnp.zeros((2, 16), jnp.bfloat16)

        def zrow(i, _):
            zbuf[pl.ds(i * 2, 2), :] = zero216
            return 0

        lax.fori_loop(0, OROWS // 2, zrow, 0)

        for b in range(nbeh):
            pltpu.sync_copy(dst_hbm.at[b, s], didx)
            for p in range(NCHUNK // NCORES):
                cc = c * (NCHUNK // NCORES) + p
                pltpu.sync_copy(srcn_hbm.at[b, cc, s], gidx)
                for z in range(RPT // OROWS):
                    pltpu.sync_copy(zbuf, acc.at[pl.ds(row0 + z * OROWS, OROWS)])
                plsc.subcore_barrier()

                def gat(j):
                    return xsflat_hbm.at[gidx.at[pl.ds(j * GBW, GBW)]]

                def sca(buf, j):
                    pltpu.sync_copy(buf, acc.at[didx.at[pl.ds(j * GBW, GBW)]],
                                    add=True)

                pltpu.async_copy(gat(0), rows0, sem0)
                pltpu.async_copy(gat(1), rows1, sem1)

                def body(i, _):
                    j0 = 2 * i
                    pltpu.make_async_copy(gat(j0), rows0, sem0).wait()
                    sca(rows0, j0)

                    @pl.when(j0 + 2 < NG)
                    def _():
                        pltpu.async_copy(gat(j0 + 2), rows0, sem0)

                    pltpu.make_async_copy(gat(j0 + 1), rows1, sem1).wait()
                    sca(rows1, j0 + 1)

                    @pl.when(j0 + 3 < NG)
                    def _():
                        pltpu.async_copy(gat(j0 + 3), rows1, sem1)

                    return 0

                lax.fori_loop(0, NG // 2, body, 0)
                pltpu.make_async_copy(gat(NG - 1), rows0, sem0).wait()
                sca(rows0, NG - 1)

                plsc.subcore_barrier()
                for z in range(RPT // OROWS):
                    pltpu.sync_copy(acc.at[pl.ds(row0 + z * OROWS, OROWS)], obuf)
                    pltpu.sync_copy(obuf, y_hbm.at[b, cc, pl.ds(row0 + z * OROWS, OROWS)])
                plsc.subcore_barrier()

    return conv


_conv_a = _make_conv(1)
_conv_b = _make_conv(3)
_conv_c = _make_conv(2)


def _pre_body(x_ref, deg_ref, o_ref):
    a = lax.rsqrt(jnp.maximum(deg_ref[:, 0:1], 1.0))
    o_ref[...] = (x_ref[...] * a).astype(jnp.bfloat16)


_pre_tc = pl.pallas_call(
    _pre_body,
    grid=(GRID,),
    in_specs=[
        pl.BlockSpec((BLK, EMB), lambda i: (i, 0)),
        pl.BlockSpec((BLK, HW), lambda i: (i, 0)),
    ],
    out_specs=pl.BlockSpec((BLK, EMB), lambda i: (i, 0)),
    out_shape=jax.ShapeDtypeStruct((NPAD, EMB), jnp.bfloat16),
)


def _post_body(y_ref, x_ref, deg_ref, o_ref):
    b = lax.rsqrt(jnp.maximum(deg_ref[:, 0:1], 1.0))
    t = y_ref[...].astype(jnp.float32) * b
    n = jnp.sqrt(jnp.sum(t * t, axis=1, keepdims=True))
    o_ref[...] = x_ref[...] + t / jnp.maximum(n, 1e-12)


_post_tc = pl.pallas_call(
    _post_body,
    grid=(GRID,),
    in_specs=[
        pl.BlockSpec((BLK, EMB), lambda i: (i, 0)),
        pl.BlockSpec((BLK, EMB), lambda i: (i, 0)),
        pl.BlockSpec((BLK, HW), lambda i: (i, 0)),
    ],
    out_specs=pl.BlockSpec((BLK, EMB), lambda i: (i, 0)),
    out_shape=jax.ShapeDtypeStruct((NPAD, EMB), jnp.float32),
)


def _fuse_body(w_ref, e0, e1, e2, e3, e4, e5, proj_ref, o_ref):
    acc = w_ref[0] * e0[...]
    for i, e in enumerate((e1, e2, e3, e4, e5)):
        acc = acc + w_ref[i + 1] * e[...]
    o_ref[...] = jnp.dot(acc, proj_ref[...],
                         preferred_element_type=jnp.float32)


_fuse_tc = pl.pallas_call(
    _fuse_body,
    grid=(GRID,),
    in_specs=[pl.BlockSpec(memory_space=pltpu.SMEM)]
    + [pl.BlockSpec((BLK, EMB), lambda i: (i, 0)) for _ in range(6)]
    + [pl.BlockSpec((EMB, EMB), lambda i: (0, 0))],
    out_specs=pl.BlockSpec((BLK, EMB), lambda i: (i, 0)),
    out_shape=jax.ShapeDtypeStruct((NPAD, EMB), jnp.float32),
)


def _conv_group(conv_fn, xs_list, packs_sel):
    nbeh = len(xs_list)
    srcn = jnp.stack([p[0] for p in packs_sel])
    dstn = jnp.stack([p[1] for p in packs_sel])
    xsflat = jnp.stack(xs_list).reshape(nbeh * NPAD * NCHUNK, CW)
    y = conv_fn(srcn, dstn, xsflat)
    return [jnp.concatenate([y[b, i] for i in range(NCHUNK)], axis=1)
            for b in range(nbeh)]


def kernel(user_table, item_table, fusion_w, fusion_proj,
           edge_ubg, edge_view, edge_cart, edge_buy,
           edge_view_buy, edge_cart_buy):
    edges = [edge_ubg, edge_view, edge_cart, edge_buy,
             edge_view_buy, edge_cart_buy]
    edges = [e.astype(jnp.int32) for e in edges]

    x0 = jnp.concatenate([user_table, item_table], axis=0)
    x0 = jnp.pad(x0, ((0, NPAD - N), (0, 0)))

    pad_ids = (N + jnp.arange(EP - E, dtype=jnp.int32) % NDUMMY)[None, :]
    pad_ids = jnp.concatenate([pad_ids, pad_ids], axis=0)
    ar = jnp.arange(NCHUNK, dtype=jnp.int32)

    def make_pack(e, beh_in_launch):
        ep = jnp.concatenate([e, pad_ids], axis=1)
        base = (beh_in_launch * NPAD + ep[0]) * NCHUNK
        srcn = base[None, :] + ar[:, None]
        return (srcn.reshape(NCHUNK, NTILES, EPT),
                ep[1].reshape(NTILES, EPT),
                ep.reshape(2, NTILES, EPT))

    packs = [make_pack(edges[0], 0), make_pack(edges[1], 0),
             make_pack(edges[2], 1), make_pack(edges[3], 2),
             make_pack(edges[4], 0), make_pack(edges[5], 1)]
    edges_all = jnp.stack([p[2] for p in packs])

    ones8 = jnp.ones((GBW, HW), jnp.float32)
    zeros8 = jnp.zeros((OROWS, HW), jnp.float32)
    degs = _hist_sc(edges_all, ones8, zeros8)

    xs0 = _pre_tc(x0, degs[0, 0])
    (y_ubg,) = _conv_group(_conv_a, [xs0], packs[0:1])
    emb_ubg = _post_tc(y_ubg, x0, degs[1, 0])

    xs_b = [_pre_tc(emb_ubg, degs[0, i]) for i in (1, 2, 3)]
    y_view, y_cart, y_buy = _conv_group(_conv_b, xs_b, packs[1:4])
    emb_view = _post_tc(y_view, emb_ubg, degs[1, 1])
    emb_cart = _post_tc(y_cart, emb_ubg, degs[1, 2])
    emb_buy = _post_tc(y_buy, emb_ubg, degs[1, 3])

    xs_c = [_pre_tc(emb_view, degs[0, 4]), _pre_tc(emb_cart, degs[0, 5])]
    y_vb, y_cb = _conv_group(_conv_c, xs_c, packs[4:6])
    emb_vb = _post_tc(y_vb, emb_view, degs[1, 4])
    emb_cb = _post_tc(y_cb, emb_cart, degs[1, 5])

    w = jax.nn.softmax(fusion_w)
    fused = _fuse_tc(w, emb_ubg, emb_view, emb_cart, emb_buy,
                     emb_vb, emb_cb, fusion_proj)
    return fused[:N]


# speedup vs baseline: 1.8988x; 1.0896x over previous
import functools

import jax
import jax.numpy as jnp
from jax import lax
from jax.experimental import pallas as pl
from jax.experimental.pallas import tpu as pltpu
from jax.experimental.pallas import tpu_sc as plsc

N_USERS = 25000
N_ITEMS = 25000
EMB = 128
E = 500000
N = (N_USERS + 1) + (N_ITEMS + 1)

NPAD = 50176
BLK = 512
GRID = NPAD // BLK

NCORES = 2
NTILES = 16
RPT = NPAD // NTILES
OROWS = RPT // 4
BW = 128
NB = 245
G = 7
NG = NB // G
GBW = G * BW
EPT = NB * BW
EP = NTILES * EPT
NCHUNK = 8
CW = EMB // NCHUNK
HW = 8
NDUMMY = NPAD - N

_MESH = plsc.VectorSubcoreMesh(
    core_axis_name="c", subcore_axis_name="s",
    num_cores=NCORES, num_subcores=NTILES)
_SC_PARAMS = pltpu.CompilerParams(use_tc_tiling_on_sc=False)


@functools.partial(
    pl.kernel,
    out_type=jax.ShapeDtypeStruct((2, 6, NPAD, HW), jnp.float32),
    mesh=_MESH,
    scratch_types=[
        pltpu.VMEM((EPT,), jnp.int32),
        pltpu.VMEM((GBW, HW), jnp.float32),
        pltpu.VMEM((OROWS, HW), jnp.float32),
        pltpu.VMEM((OROWS, HW), jnp.float32),
        pltpu.VMEM_SHARED((NPAD, HW), jnp.float32),
    ],
    compiler_params=_SC_PARAMS,
)
def _hist_sc(edges_hbm, ones_hbm, zeros_hbm, degs_hbm, ids, ones, zbuf, obuf, acc):
    c = lax.axis_index("c")
    s = lax.axis_index("s")
    row0 = s * RPT

    pltpu.sync_copy(ones_hbm, ones)
    pltpu.sync_copy(zeros_hbm, zbuf)

    for b in range(6):
        pltpu.sync_copy(edges_hbm.at[b, c, s], ids)
        for z in range(RPT // OROWS):
            pltpu.sync_copy(zbuf, acc.at[pl.ds(row0 + z * OROWS, OROWS)])
        plsc.subcore_barrier()

        def body(j, _):
            pltpu.sync_copy(ones, acc.at[ids.at[pl.ds(j * GBW, GBW)]], add=True)
            return 0

        lax.fori_loop(0, NG, body, 0)
        plsc.subcore_barrier()
        for z in range(RPT // OROWS):
            pltpu.sync_copy(acc.at[pl.ds(row0 + z * OROWS, OROWS)], obuf)
            pltpu.sync_copy(obuf, degs_hbm.at[c, b, pl.ds(row0 + z * OROWS, OROWS)])
        plsc.subcore_barrier()


def _make_conv(nbeh):
    @functools.partial(
        pl.kernel,
        out_type=jax.ShapeDtypeStruct((nbeh, NCHUNK, NPAD, CW), jnp.bfloat16),
        mesh=_MESH,
        scratch_types=[
            pltpu.VMEM((EPT,), jnp.int32),
            pltpu.VMEM((EPT,), jnp.int32),
            pltpu.VMEM((GBW, CW), jnp.bfloat16),
            pltpu.VMEM((GBW, CW), jnp.bfloat16),
            pltpu.VMEM((OROWS, CW), jnp.bfloat16),
            pltpu.VMEM((OROWS, CW), jnp.bfloat16),
            pltpu.VMEM_SHARED((NPAD, CW), jnp.bfloat16),
            pltpu.SemaphoreType.DMA,
            pltpu.SemaphoreType.DMA,
        ],
        compiler_params=_SC_PARAMS,
    )
    def conv(srcn_hbm, dst_hbm, xsflat_hbm, y_hbm,
             gidx, didx, rows0, rows1, zbuf, obuf, acc, sem0, sem1):
        c = lax.axis_index("c")
        s = lax.axis_index("s")
        row0 = s * RPT

        zero216 = jnp.zeros((2, 16), jnp.bfloat16)

        def zrow(i, _):
            zbuf[pl.ds(i * 2, 2), :] = zero216
            return 0

        lax.fori_loop(0, OROWS // 2, zrow, 0)

        for b in range(nbeh):
            pltpu.sync_copy(dst_hbm.at[b, s], didx)
            for p in range(NCHUNK // NCORES):
                cc = c * (NCHUNK // NCORES) + p
                pltpu.sync_copy(srcn_hbm.at[b, cc, s], gidx)
                for z in range(RPT // OROWS):
                    pltpu.sync_copy(zbuf, acc.at[pl.ds(row0 + z * OROWS, OROWS)])
                plsc.subcore_barrier()

                def gat(j):
                    return xsflat_hbm.at[gidx.at[pl.ds(j * GBW, GBW)]]

                def sca(buf, j):
                    pltpu.sync_copy(buf, acc.at[didx.at[pl.ds(j * GBW, GBW)]],
                                    add=True)

                pltpu.async_copy(gat(0), rows0, sem0)
                pltpu.async_copy(gat(1), rows1, sem1)

                def body(i, _):
                    j0 = 2 * i
                    pltpu.make_async_copy(gat(j0), rows0, sem0).wait()
                    sca(rows0, j0)

                    @pl.when(j0 + 2 < NG)
                    def _():
                        pltpu.async_copy(gat(j0 + 2), rows0, sem0)

                    pltpu.make_async_copy(gat(j0 + 1), rows1, sem1).wait()
                    sca(rows1, j0 + 1)

                    @pl.when(j0 + 3 < NG)
                    def _():
                        pltpu.async_copy(gat(j0 + 3), rows1, sem1)

                    return 0

                lax.fori_loop(0, NG // 2, body, 0)
                pltpu.make_async_copy(gat(NG - 1), rows0, sem0).wait()
                sca(rows0, NG - 1)

                plsc.subcore_barrier()
                for z in range(RPT // OROWS):
                    pltpu.sync_copy(acc.at[pl.ds(row0 + z * OROWS, OROWS)], obuf)
                    pltpu.sync_copy(obuf, y_hbm.at[b, cc, pl.ds(row0 + z * OROWS, OROWS)])
                plsc.subcore_barrier()

    return conv


_conv_a = _make_conv(1)
_conv_b = _make_conv(3)
_conv_c = _make_conv(2)


def _pre_body(x_ref, deg_ref, o_ref):
    a = lax.rsqrt(jnp.maximum(deg_ref[:, 0:1], 1.0))
    o_ref[...] = (x_ref[...] * a).astype(jnp.bfloat16)


_pre_tc = pl.pallas_call(
    _pre_body,
    grid=(GRID,),
    in_specs=[
        pl.BlockSpec((BLK, EMB), lambda i: (i, 0)),
        pl.BlockSpec((BLK, HW), lambda i: (i, 0)),
    ],
    out_specs=pl.BlockSpec((BLK, EMB), lambda i: (i, 0)),
    out_shape=jax.ShapeDtypeStruct((NPAD, EMB), jnp.bfloat16),
)


def _post_body(y_ref, x_ref, deg_ref, o_ref):
    b = lax.rsqrt(jnp.maximum(deg_ref[:, 0:1], 1.0))
    y = jnp.concatenate([y_ref[k] for k in range(NCHUNK)], axis=1)
    t = y.astype(jnp.float32) * b
    n = jnp.sqrt(jnp.sum(t * t, axis=1, keepdims=True))
    o_ref[...] = x_ref[...] + t / jnp.maximum(n, 1e-12)


_post_tc = pl.pallas_call(
    _post_body,
    grid=(GRID,),
    in_specs=[
        pl.BlockSpec((NCHUNK, BLK, CW), lambda i: (0, i, 0)),
        pl.BlockSpec((BLK, EMB), lambda i: (i, 0)),
        pl.BlockSpec((BLK, HW), lambda i: (i, 0)),
    ],
    out_specs=pl.BlockSpec((BLK, EMB), lambda i: (i, 0)),
    out_shape=jax.ShapeDtypeStruct((NPAD, EMB), jnp.float32),
)


def _fuse_body(w_ref, e0, e1, e2, e3, e4, e5, proj_ref, o_ref):
    acc = w_ref[0] * e0[...]
    for i, e in enumerate((e1, e2, e3, e4, e5)):
        acc = acc + w_ref[i + 1] * e[...]
    o_ref[...] = jnp.dot(acc, proj_ref[...],
                         preferred_element_type=jnp.float32)


_fuse_tc = pl.pallas_call(
    _fuse_body,
    grid=(GRID,),
    in_specs=[pl.BlockSpec(memory_space=pltpu.SMEM)]
    + [pl.BlockSpec((BLK, EMB), lambda i: (i, 0)) for _ in range(6)]
    + [pl.BlockSpec((EMB, EMB), lambda i: (0, 0))],
    out_specs=pl.BlockSpec((BLK, EMB), lambda i: (i, 0)),
    out_shape=jax.ShapeDtypeStruct((NPAD, EMB), jnp.float32),
)


def _conv_group(conv_fn, xs_list, packs_sel):
    nbeh = len(xs_list)
    srcn = jnp.stack([p[0] for p in packs_sel])
    dstn = jnp.stack([p[1] for p in packs_sel])
    xsflat = jnp.stack(xs_list).reshape(nbeh * NPAD * NCHUNK, CW)
    y = conv_fn(srcn, dstn, xsflat)
    return [y[b] for b in range(nbeh)]


def kernel(user_table, item_table, fusion_w, fusion_proj,
           edge_ubg, edge_view, edge_cart, edge_buy,
           edge_view_buy, edge_cart_buy):
    edges = [edge_ubg, edge_view, edge_cart, edge_buy,
             edge_view_buy, edge_cart_buy]
    edges = [e.astype(jnp.int32) for e in edges]

    x0 = jnp.concatenate([user_table, item_table], axis=0)
    x0 = jnp.pad(x0, ((0, NPAD - N), (0, 0)))

    pad_ids = (N + jnp.arange(EP - E, dtype=jnp.int32) % NDUMMY)[None, :]
    pad_ids = jnp.concatenate([pad_ids, pad_ids], axis=0)
    ar = jnp.arange(NCHUNK, dtype=jnp.int32)

    def make_pack(e, beh_in_launch):
        ep = jnp.concatenate([e, pad_ids], axis=1)
        base = (beh_in_launch * NPAD + ep[0]) * NCHUNK
        srcn = base[None, :] + ar[:, None]
        return (srcn.reshape(NCHUNK, NTILES, EPT),
                ep[1].reshape(NTILES, EPT),
                ep.reshape(2, NTILES, EPT))

    packs = [make_pack(edges[0], 0), make_pack(edges[1], 0),
             make_pack(edges[2], 1), make_pack(edges[3], 2),
             make_pack(edges[4], 0), make_pack(edges[5], 1)]
    edges_all = jnp.stack([p[2] for p in packs])

    ones8 = jnp.ones((GBW, HW), jnp.float32)
    zeros8 = jnp.zeros((OROWS, HW), jnp.float32)
    degs = _hist_sc(edges_all, ones8, zeros8)

    xs0 = _pre_tc(x0, degs[0, 0])
    (y_ubg,) = _conv_group(_conv_a, [xs0], packs[0:1])
    emb_ubg = _post_tc(y_ubg, x0, degs[1, 0])

    xs_b = [_pre_tc(emb_ubg, degs[0, i]) for i in (1, 2, 3)]
    y_view, y_cart, y_buy = _conv_group(_conv_b, xs_b, packs[1:4])
    emb_view = _post_tc(y_view, emb_ubg, degs[1, 1])
    emb_cart = _post_tc(y_cart, emb_ubg, degs[1, 2])
    emb_buy = _post_tc(y_buy, emb_ubg, degs[1, 3])

    xs_c = [_pre_tc(emb_view, degs[0, 4]), _pre_tc(emb_cart, degs[0, 5])]
    y_vb, y_cb = _conv_group(_conv_c, xs_c, packs[4:6])
    emb_vb = _post_tc(y_vb, emb_view, degs[1, 4])
    emb_cb = _post_tc(y_cb, emb_cart, degs[1, 5])

    w = jax.nn.softmax(fusion_w)
    fused = _fuse_tc(w, emb_ubg, emb_view, emb_cart, emb_buy,
                     emb_vb, emb_cb, fusion_proj)
    return fused[:N]


# speedup vs baseline: 1.9108x; 1.0063x over previous
import functools

import jax
import jax.numpy as jnp
from jax import lax
from jax.experimental import pallas as pl
from jax.experimental.pallas import tpu as pltpu
from jax.experimental.pallas import tpu_sc as plsc

N_USERS = 25000
N_ITEMS = 25000
EMB = 128
E = 500000
N = (N_USERS + 1) + (N_ITEMS + 1)

NPAD = 50176
BLK = 512
GRID = NPAD // BLK

NCORES = 2
NTILES = 16
RPT = NPAD // NTILES
OROWS = RPT // 4
BW = 128
NB = 245
G = 7
NG = NB // G
GBW = G * BW
EPT = NB * BW
EP = NTILES * EPT
NCHUNK = 8
CW = EMB // NCHUNK
HW = 8
NDUMMY = NPAD - N

_MESH = plsc.VectorSubcoreMesh(
    core_axis_name="c", subcore_axis_name="s",
    num_cores=NCORES, num_subcores=NTILES)
_SC_PARAMS = pltpu.CompilerParams(use_tc_tiling_on_sc=False)


@functools.partial(
    pl.kernel,
    out_type=jax.ShapeDtypeStruct((2, 6, NPAD, HW), jnp.float32),
    mesh=_MESH,
    scratch_types=[
        pltpu.VMEM((EPT,), jnp.int32),
        pltpu.VMEM((GBW, HW), jnp.float32),
        pltpu.VMEM((OROWS, HW), jnp.float32),
        pltpu.VMEM((OROWS, HW), jnp.float32),
        pltpu.VMEM_SHARED((NPAD, HW), jnp.float32),
    ],
    compiler_params=_SC_PARAMS,
)
def _hist_sc(edges_hbm, ones_hbm, zeros_hbm, degs_hbm, ids, ones, zbuf, obuf, acc):
    c = lax.axis_index("c")
    s = lax.axis_index("s")
    row0 = s * RPT

    pltpu.sync_copy(ones_hbm, ones)
    pltpu.sync_copy(zeros_hbm, zbuf)

    for b in range(6):
        pltpu.sync_copy(edges_hbm.at[b, c, s], ids)
        for z in range(RPT // OROWS):
            pltpu.sync_copy(zbuf, acc.at[pl.ds(row0 + z * OROWS, OROWS)])
        plsc.subcore_barrier()

        def body(j, _):
            pltpu.sync_copy(ones, acc.at[ids.at[pl.ds(j * GBW, GBW)]], add=True)
            return 0

        lax.fori_loop(0, NG, body, 0)
        plsc.subcore_barrier()
        for z in range(RPT // OROWS):
            pltpu.sync_copy(acc.at[pl.ds(row0 + z * OROWS, OROWS)], obuf)
            pltpu.sync_copy(obuf, degs_hbm.at[c, b, pl.ds(row0 + z * OROWS, OROWS)])
        plsc.subcore_barrier()


def _make_conv(nbeh):
    @functools.partial(
        pl.kernel,
        out_type=jax.ShapeDtypeStruct((nbeh, NCHUNK, NPAD, CW), jnp.bfloat16),
        mesh=_MESH,
        scratch_types=[
            pltpu.VMEM((EPT,), jnp.int32),
            pltpu.VMEM((EPT,), jnp.int32),
            pltpu.VMEM((GBW, CW), jnp.bfloat16),
            pltpu.VMEM((GBW, CW), jnp.bfloat16),
            pltpu.VMEM((OROWS, CW), jnp.bfloat16),
            pltpu.VMEM((OROWS, CW), jnp.bfloat16),
            pltpu.VMEM_SHARED((NPAD, CW), jnp.bfloat16),
            pltpu.SemaphoreType.DMA,
            pltpu.SemaphoreType.DMA,
        ],
        compiler_params=_SC_PARAMS,
    )
    def conv(srcn_hbm, dst_hbm, *rest):
        xs_refs = rest[:nbeh]
        (y_hbm, gidx, didx, rows0, rows1, zbuf, obuf, acc, sem0, sem1) = rest[nbeh:]
        c = lax.axis_index("c")
        s = lax.axis_index("s")
        row0 = s * RPT

        zero216 = jnp.zeros((2, 16), jnp.bfloat16)

        def zrow(i, _):
            zbuf[pl.ds(i * 2, 2), :] = zero216
            return 0

        lax.fori_loop(0, OROWS // 2, zrow, 0)

        for b in range(nbeh):
            xsflat_hbm = xs_refs[b]
            pltpu.sync_copy(dst_hbm.at[b, s], didx)
            for p in range(NCHUNK // NCORES):
                cc = c * (NCHUNK // NCORES) + p
                pltpu.sync_copy(srcn_hbm.at[b, cc, s], gidx)
                for z in range(RPT // OROWS):
                    pltpu.sync_copy(zbuf, acc.at[pl.ds(row0 + z * OROWS, OROWS)])
                plsc.subcore_barrier()

                def gat(j):
                    return xsflat_hbm.at[gidx.at[pl.ds(j * GBW, GBW)]]

                def sca(buf, j):
                    pltpu.sync_copy(buf, acc.at[didx.at[pl.ds(j * GBW, GBW)]],
                                    add=True)

                pltpu.async_copy(gat(0), rows0, sem0)
                pltpu.async_copy(gat(1), rows1, sem1)

                def body(i, _):
                    j0 = 2 * i
                    pltpu.make_async_copy(gat(j0), rows0, sem0).wait()
                    sca(rows0, j0)

                    @pl.when(j0 + 2 < NG)
                    def _():
                        pltpu.async_copy(gat(j0 + 2), rows0, sem0)

                    pltpu.make_async_copy(gat(j0 + 1), rows1, sem1).wait()
                    sca(rows1, j0 + 1)

                    @pl.when(j0 + 3 < NG)
                    def _():
                        pltpu.async_copy(gat(j0 + 3), rows1, sem1)

                    return 0

                lax.fori_loop(0, NG // 2, body, 0)
                pltpu.make_async_copy(gat(NG - 1), rows0, sem0).wait()
                sca(rows0, NG - 1)

                plsc.subcore_barrier()
                for z in range(RPT // OROWS):
                    pltpu.sync_copy(acc.at[pl.ds(row0 + z * OROWS, OROWS)], obuf)
                    pltpu.sync_copy(obuf, y_hbm.at[b, cc, pl.ds(row0 + z * OROWS, OROWS)])
                plsc.subcore_barrier()

    return conv


_conv_a = _make_conv(1)
_conv_b = _make_conv(3)
_conv_c = _make_conv(2)


def _pre_body(x_ref, deg_ref, o_ref):
    a = lax.rsqrt(jnp.maximum(deg_ref[:, 0:1], 1.0))
    o_ref[...] = (x_ref[...] * a).astype(jnp.bfloat16)


_pre_tc = pl.pallas_call(
    _pre_body,
    grid=(GRID,),
    in_specs=[
        pl.BlockSpec((BLK, EMB), lambda i: (i, 0)),
        pl.BlockSpec((BLK, HW), lambda i: (i, 0)),
    ],
    out_specs=pl.BlockSpec((BLK, EMB), lambda i: (i, 0)),
    out_shape=jax.ShapeDtypeStruct((NPAD, EMB), jnp.bfloat16),
)


def _post_body(y_ref, x_ref, deg_ref, o_ref):
    b = lax.rsqrt(jnp.maximum(deg_ref[:, 0:1], 1.0))
    y = jnp.concatenate([y_ref[k] for k in range(NCHUNK)], axis=1)
    t = y.astype(jnp.float32) * b
    n = jnp.sqrt(jnp.sum(t * t, axis=1, keepdims=True))
    o_ref[...] = x_ref[...] + t / jnp.maximum(n, 1e-12)


_post_tc = pl.pallas_call(
    _post_body,
    grid=(GRID,),
    in_specs=[
        pl.BlockSpec((NCHUNK, BLK, CW), lambda i: (0, i, 0)),
        pl.BlockSpec((BLK, EMB), lambda i: (i, 0)),
        pl.BlockSpec((BLK, HW), lambda i: (i, 0)),
    ],
    out_specs=pl.BlockSpec((BLK, EMB), lambda i: (i, 0)),
    out_shape=jax.ShapeDtypeStruct((NPAD, EMB), jnp.float32),
)


def _fuse_body(w_ref, e0, e1, e2, e3, e4, e5, proj_ref, o_ref):
    acc = w_ref[0] * e0[...]
    for i, e in enumerate((e1, e2, e3, e4, e5)):
        acc = acc + w_ref[i + 1] * e[...]
    o_ref[...] = jnp.dot(acc, proj_ref[...],
                         preferred_element_type=jnp.float32)


_fuse_tc = pl.pallas_call(
    _fuse_body,
    grid=(GRID,),
    in_specs=[pl.BlockSpec(memory_space=pltpu.SMEM)]
    + [pl.BlockSpec((BLK, EMB), lambda i: (i, 0)) for _ in range(6)]
    + [pl.BlockSpec((EMB, EMB), lambda i: (0, 0))],
    out_specs=pl.BlockSpec((BLK, EMB), lambda i: (i, 0)),
    out_shape=jax.ShapeDtypeStruct((NPAD, EMB), jnp.float32),
)


def _conv_group(conv_fn, xs_list, packs_sel):
    nbeh = len(xs_list)
    srcn = jnp.stack([p[0] for p in packs_sel])
    dstn = jnp.stack([p[1] for p in packs_sel])
    xs_flat = [x.reshape(NPAD * NCHUNK, CW) for x in xs_list]
    y = conv_fn(srcn, dstn, *xs_flat)
    return [y[b] for b in range(nbeh)]


def kernel(user_table, item_table, fusion_w, fusion_proj,
           edge_ubg, edge_view, edge_cart, edge_buy,
           edge_view_buy, edge_cart_buy):
    edges = [edge_ubg, edge_view, edge_cart, edge_buy,
             edge_view_buy, edge_cart_buy]
    edges = [e.astype(jnp.int32) for e in edges]

    x0 = jnp.concatenate([user_table, item_table], axis=0)
    x0 = jnp.pad(x0, ((0, NPAD - N), (0, 0)))

    pad_ids = (N + jnp.arange(EP - E, dtype=jnp.int32) % NDUMMY)[None, :]
    pad_ids = jnp.concatenate([pad_ids, pad_ids], axis=0)
    ar = jnp.arange(NCHUNK, dtype=jnp.int32)

    def make_pack(e):
        ep = jnp.concatenate([e, pad_ids], axis=1)
        srcn = (ep[0] * NCHUNK)[None, :] + ar[:, None]
        return (srcn.reshape(NCHUNK, NTILES, EPT),
                ep[1].reshape(NTILES, EPT),
                ep.reshape(2, NTILES, EPT))

    packs = [make_pack(e) for e in edges]
    edges_all = jnp.stack([p[2] for p in packs])

    ones8 = jnp.ones((GBW, HW), jnp.float32)
    zeros8 = jnp.zeros((OROWS, HW), jnp.float32)
    degs = _hist_sc(edges_all, ones8, zeros8)

    xs0 = _pre_tc(x0, degs[0, 0])
    (y_ubg,) = _conv_group(_conv_a, [xs0], packs[0:1])
    emb_ubg = _post_tc(y_ubg, x0, degs[1, 0])

    xs_b = [_pre_tc(emb_ubg, degs[0, i]) for i in (1, 2, 3)]
    y_view, y_cart, y_buy = _conv_group(_conv_b, xs_b, packs[1:4])
    emb_view = _post_tc(y_view, emb_ubg, degs[1, 1])
    emb_cart = _post_tc(y_cart, emb_ubg, degs[1, 2])
    emb_buy = _post_tc(y_buy, emb_ubg, degs[1, 3])

    xs_c = [_pre_tc(emb_view, degs[0, 4]), _pre_tc(emb_cart, degs[0, 5])]
    y_vb, y_cb = _conv_group(_conv_c, xs_c, packs[4:6])
    emb_vb = _post_tc(y_vb, emb_view, degs[1, 4])
    emb_cb = _post_tc(y_cb, emb_cart, degs[1, 5])

    w = jax.nn.softmax(fusion_w)
    fused = _fuse_tc(w, emb_ubg, emb_view, emb_cart, emb_buy,
                     emb_vb, emb_cb, fusion_proj)
    return fused[:N]
